# baseline (device time: 44569 ns/iter reference)
import jax
import jax.numpy as jnp
from jax import lax
from jax.experimental import pallas as pl
from jax.experimental.pallas import tpu as pltpu

N_DEV = 4
N_CHUNKS = 8


def kernel(t):
    m, n = t.shape
    half = m // 2
    quart = m // 4
    w = n // N_CHUNKS

    def body(t_ref, out_ref, *scratch):
        send1 = scratch[0:N_CHUNKS]
        recv1 = scratch[N_CHUNKS:2 * N_CHUNKS]
        acc = scratch[2 * N_CHUNKS:3 * N_CHUNKS]
        recv2 = scratch[3 * N_CHUNKS:4 * N_CHUNKS]
        send_sems, recv_sems = scratch[4 * N_CHUNKS:]

        my = lax.axis_index("i")
        ybit = (my ^ (my >> 1)) & 1
        xbit = my >> 1
        yp = my ^ 1
        xp = 3 - my

        cfg = []
        for c in range(N_CHUNKS):
            if c % 2 == 0:
                cfg.append((c * w, yp, ybit, xp, xbit))
            else:
                cfg.append((c * w, xp, xbit, yp, ybit))

        def rdma(src, dst, slot, dev):
            return pltpu.make_async_remote_copy(
                src_ref=src, dst_ref=dst,
                send_sem=send_sems.at[slot], recv_sem=recv_sems.at[slot],
                device_id=(dev,), device_id_type=pl.DeviceIdType.MESH,
            )

        def f(s):
            r = jnp.maximum(s, 0.0)
            return jnp.tanh(s) * s * s + r * r * r

        barrier_sem = pltpu.get_barrier_semaphore()
        for nbr in (yp, xp):
            pl.semaphore_signal(
                barrier_sem, inc=1,
                device_id=(nbr,), device_id_type=pl.DeviceIdType.MESH,
            )
        pl.semaphore_wait(barrier_sem, 2)

        r1 = []
        for c, (col0, p1, k1, _, _) in enumerate(cfg):
            send1[c][:, :] = t_ref[
                pl.ds((1 - k1) * half, half), pl.ds(col0, w)
            ].astype(jnp.bfloat16)
            r = rdma(send1[c], recv1[c], c, p1)
            r.start()
            r1.append(r)

        r2 = []
        for c, (col0, p1, k1, p2, k2) in enumerate(cfg):
            r1[c].wait()
            acc[c][:, :] = (
                t_ref[pl.ds(k1 * half, half), pl.ds(col0, w)]
                + recv1[c][:, :].astype(jnp.float32)
            ).astype(jnp.bfloat16)
            r = rdma(acc[c].at[pl.ds((1 - k2) * quart, quart), :],
                     recv2[c], N_CHUNKS + c, p2)
            r.start()
            r2.append(r)

        r3 = []
        for c, (col0, p1, k1, p2, k2) in enumerate(cfg):
            r2[c].wait()
            s = (
                acc[c][pl.ds(k2 * quart, quart), :].astype(jnp.float32)
                + recv2[c][:, :].astype(jnp.float32)
            )
            q0 = k1 * half + k2 * quart
            out_ref[pl.ds(q0, quart), pl.ds(col0, w)] = f(s).astype(jnp.bfloat16)
            r = rdma(out_ref.at[pl.ds(q0, quart), pl.ds(col0, w)],
                     out_ref.at[pl.ds(q0, quart), pl.ds(col0, w)],
                     2 * N_CHUNKS + c, p2)
            r.start()
            r3.append(r)

        r4 = []
        for c, (col0, p1, k1, p2, k2) in enumerate(cfg):
            r3[c].wait()
            r = rdma(out_ref.at[pl.ds(k1 * half, half), pl.ds(col0, w)],
                     out_ref.at[pl.ds(k1 * half, half), pl.ds(col0, w)],
                     3 * N_CHUNKS + c, p1)
            r.start()
            r4.append(r)
        for c in range(N_CHUNKS):
            r4[c].wait()

    return pl.pallas_call(
        body,
        out_shape=jax.ShapeDtypeStruct((m, n), jnp.bfloat16),
        in_specs=[pl.BlockSpec(memory_space=pltpu.VMEM)],
        out_specs=pl.BlockSpec(memory_space=pltpu.VMEM),
        scratch_shapes=(
            [pltpu.VMEM((half, w), jnp.bfloat16)] * N_CHUNKS
            + [pltpu.VMEM((half, w), jnp.bfloat16)] * N_CHUNKS
            + [pltpu.VMEM((half, w), jnp.bfloat16)] * N_CHUNKS
            + [pltpu.VMEM((quart, w), jnp.bfloat16)] * N_CHUNKS
            + [
                pltpu.SemaphoreType.DMA((4 * N_CHUNKS,)),
                pltpu.SemaphoreType.DMA((4 * N_CHUNKS,)),
            ]
        ),
        compiler_params=pltpu.CompilerParams(collective_id=0),
    )(t)


# device time: 44333 ns/iter; 1.0053x vs baseline; 1.0053x over previous
import jax
import jax.numpy as jnp
from jax import lax
from jax.experimental import pallas as pl
from jax.experimental.pallas import tpu as pltpu

N_DEV = 4
N_CHUNKS = 4


def kernel(t):
    m, n = t.shape
    half = m // 2
    quart = m // 4
    w = n // N_CHUNKS

    def body(t_ref, out_ref, *scratch):
        send1 = scratch[0:N_CHUNKS]
        recv1 = scratch[N_CHUNKS:2 * N_CHUNKS]
        acc_s = scratch[2 * N_CHUNKS:3 * N_CHUNKS]
        acc_k = scratch[3 * N_CHUNKS:4 * N_CHUNKS]
        recv2 = scratch[4 * N_CHUNKS:5 * N_CHUNKS]
        send_sems, recv_sems = scratch[5 * N_CHUNKS:]

        my = lax.axis_index("i")
        ybit = (my ^ (my >> 1)) & 1
        xbit = my >> 1
        yp = my ^ 1
        xp = 3 - my

        cfg = []
        for c in range(N_CHUNKS):
            if c % 2 == 0:
                cfg.append((c * w, yp, ybit, xp, xbit))
            else:
                cfg.append((c * w, xp, xbit, yp, ybit))

        def rdma(src, dst, slot, dev):
            return pltpu.make_async_remote_copy(
                src_ref=src, dst_ref=dst,
                send_sem=send_sems.at[slot], recv_sem=recv_sems.at[slot],
                device_id=(dev,), device_id_type=pl.DeviceIdType.MESH,
            )

        def f(s):
            r = jnp.maximum(s, 0.0)
            return jnp.tanh(s) * s * s + r * r * r

        barrier_sem = pltpu.get_barrier_semaphore()
        for nbr in (yp, xp):
            pl.semaphore_signal(
                barrier_sem, inc=1,
                device_id=(nbr,), device_id_type=pl.DeviceIdType.MESH,
            )
        pl.semaphore_wait(barrier_sem, 2)

        r1 = []
        for c, (col0, p1, k1, _, _) in enumerate(cfg):
            send1[c][:, :] = t_ref[
                pl.ds((1 - k1) * half, half), pl.ds(col0, w)
            ].astype(jnp.bfloat16)
            r = rdma(send1[c], recv1[c], c, p1)
            r.start()
            r1.append(r)

        r2 = []
        for c, (col0, p1, k1, p2, k2) in enumerate(cfg):
            r1[c].wait()
            acc_s[c][:, :] = (
                t_ref[pl.ds(k1 * half + (1 - k2) * quart, quart), pl.ds(col0, w)]
                + recv1[c][pl.ds((1 - k2) * quart, quart), :].astype(jnp.float32)
            ).astype(jnp.bfloat16)
            r = rdma(acc_s[c], recv2[c], N_CHUNKS + c, p2)
            r.start()
            r2.append(r)
            acc_k[c][:, :] = (
                t_ref[pl.ds(k1 * half + k2 * quart, quart), pl.ds(col0, w)]
                + recv1[c][pl.ds(k2 * quart, quart), :].astype(jnp.float32)
            ).astype(jnp.bfloat16)

        r3, r4a = [], []
        for c, (col0, p1, k1, p2, k2) in enumerate(cfg):
            r2[c].wait()
            s = (
                acc_k[c][:, :].astype(jnp.float32)
                + recv2[c][:, :].astype(jnp.float32)
            )
            q0 = k1 * half + k2 * quart
            out_ref[pl.ds(q0, quart), pl.ds(col0, w)] = f(s).astype(jnp.bfloat16)
            r = rdma(out_ref.at[pl.ds(q0, quart), pl.ds(col0, w)],
                     out_ref.at[pl.ds(q0, quart), pl.ds(col0, w)],
                     2 * N_CHUNKS + c, p2)
            r.start()
            r3.append(r)
            r = rdma(out_ref.at[pl.ds(q0, quart), pl.ds(col0, w)],
                     out_ref.at[pl.ds(q0, quart), pl.ds(col0, w)],
                     3 * N_CHUNKS + c, p1)
            r.start()
            r4a.append(r)

        r4b = []
        for c, (col0, p1, k1, p2, k2) in enumerate(cfg):
            r3[c].wait()
            q1 = k1 * half + (1 - k2) * quart
            r = rdma(out_ref.at[pl.ds(q1, quart), pl.ds(col0, w)],
                     out_ref.at[pl.ds(q1, quart), pl.ds(col0, w)],
                     4 * N_CHUNKS + c, p1)
            r.start()
            r4b.append(r)
        for c in range(N_CHUNKS):
            r4a[c].wait()
            r4b[c].wait()

    return pl.pallas_call(
        body,
        out_shape=jax.ShapeDtypeStruct((m, n), jnp.bfloat16),
        in_specs=[pl.BlockSpec(memory_space=pltpu.VMEM)],
        out_specs=pl.BlockSpec(memory_space=pltpu.VMEM),
        scratch_shapes=(
            [pltpu.VMEM((half, w), jnp.bfloat16)] * N_CHUNKS
            + [pltpu.VMEM((half, w), jnp.bfloat16)] * N_CHUNKS
            + [pltpu.VMEM((quart, w), jnp.bfloat16)] * N_CHUNKS
            + [pltpu.VMEM((quart, w), jnp.bfloat16)] * N_CHUNKS
            + [pltpu.VMEM((quart, w), jnp.bfloat16)] * N_CHUNKS
            + [
                pltpu.SemaphoreType.DMA((5 * N_CHUNKS,)),
                pltpu.SemaphoreType.DMA((5 * N_CHUNKS,)),
            ]
        ),
        compiler_params=pltpu.CompilerParams(collective_id=0),
    )(t)
